# baseline (speedup 1.0000x reference)
import jax
import jax.numpy as jnp
from jax import lax
from jax.experimental import pallas as pl
from jax.experimental.pallas import tpu as pltpu
from jax.experimental.pallas import tpu_sc as plsc

_VOCAB = 1000000
_D = 64
_T = 128
_B = 4096
_S = 200

_NC = 2
_NS = 16
_NW = _NC * _NS
_BPW = _B // _NW


def _sc_pool_body(xf_hbm, table_hbm, out_hbm, xb_v, ibase_v, iota_v,
                  idxc0, idxc1, r0, r1, acc_sh, sem0, sem1):
    sid = lax.axis_index("s")
    wid = sid * _NC + lax.axis_index("c")
    base = wid * _BPW

    for k in range(_BPW // 16):
        iv = lax.iota(jnp.int32, 16) + 16 * k
        iota_v[pl.ds(16 * k, 16)] = iv + sid * _BPW
        ibase_v[pl.ds(16 * k, 16)] = iv

    pltpu.sync_copy(xf_hbm.at[pl.ds(base, _BPW)], xb_v)

    def fill_idx(dst, j):
        for k in range(_BPW // 16):
            rows = ibase_v[pl.ds(16 * k, 16)]
            cols = lax.iota(jnp.int32, 16) * 0 + j
            dst[pl.ds(16 * k, 16)] = plsc.load_gather(xb_v, [rows, cols])

    fill_idx(idxc0, 0)
    pltpu.async_copy(table_hbm.at[idxc0], r0, sem0).wait()
    fill_idx(idxc1, 1)
    pltpu.async_copy(table_hbm.at[idxc1], r1, sem1)
    pltpu.sync_copy(r0, acc_sh.at[iota_v])

    @pl.loop(2, _S, step=2)
    def _(j):
        fill_idx(idxc0, j)
        pltpu.async_copy(table_hbm.at[idxc0], r0, sem0)
        pltpu.make_async_copy(table_hbm.at[idxc1], r1, sem1).wait()
        pltpu.sync_copy(r1, acc_sh.at[iota_v], add=True)
        fill_idx(idxc1, j + 1)
        pltpu.async_copy(table_hbm.at[idxc1], r1, sem1)
        pltpu.make_async_copy(table_hbm.at[idxc0], r0, sem0).wait()
        pltpu.sync_copy(r0, acc_sh.at[iota_v], add=True)

    pltpu.make_async_copy(table_hbm.at[idxc1], r1, sem1).wait()
    pltpu.sync_copy(r1, acc_sh.at[iota_v], add=True)

    pltpu.sync_copy(acc_sh.at[pl.ds(sid * _BPW, _BPW)],
                    out_hbm.at[pl.ds(base, _BPW)])


def _sc_pool(xf, emb_table):
    mesh = plsc.VectorSubcoreMesh(core_axis_name="c", subcore_axis_name="s")
    return pl.kernel(
        _sc_pool_body,
        out_type=jax.ShapeDtypeStruct((_B, _D), jnp.float32),
        mesh=mesh,
        scratch_types=[
            pltpu.VMEM((_BPW, _S), jnp.int32),
            pltpu.VMEM((_BPW,), jnp.int32),
            pltpu.VMEM((_BPW,), jnp.int32),
            pltpu.VMEM((_BPW,), jnp.int32),
            pltpu.VMEM((_BPW,), jnp.int32),
            pltpu.VMEM((_BPW, _D), jnp.float32),
            pltpu.VMEM((_BPW, _D), jnp.float32),
            pltpu.VMEM_SHARED((_NS * _BPW, _D), jnp.float32),
            pltpu.SemaphoreType.DMA,
            pltpu.SemaphoreType.DMA,
        ],
        compiler_params=pltpu.CompilerParams(
            use_tc_tiling_on_sc=False, needs_layout_passes=False),
    )(xf, emb_table)


def _mm_body(p_ref, w_ref, b_ref, o_ref):
    o_ref[...] = (
        jnp.dot(p_ref[...], w_ref[...],
                preferred_element_type=jnp.float32,
                precision=lax.Precision.HIGHEST)
        + b_ref[...]
    )


def _tc_project(pooled, W, b):
    blk = 512
    return pl.pallas_call(
        _mm_body,
        out_shape=jax.ShapeDtypeStruct((_B, _T), jnp.float32),
        grid=(_B // blk,),
        in_specs=[
            pl.BlockSpec((blk, _D), lambda i: (i, 0)),
            pl.BlockSpec((_D, _T), lambda i: (0, 0)),
            pl.BlockSpec((1, _T), lambda i: (0, 0)),
        ],
        out_specs=pl.BlockSpec((blk, _T), lambda i: (i, 0)),
    )(pooled, W, b.reshape(1, _T))


def kernel(x, emb_table, W, b):
    pooled = _sc_pool(x, emb_table)
    return _tc_project(pooled, W, b)


# speedup vs baseline: 1.8567x; 1.8567x over previous
import jax
import jax.numpy as jnp
from jax import lax
from jax.experimental import pallas as pl
from jax.experimental.pallas import tpu as pltpu
from jax.experimental.pallas import tpu_sc as plsc

_VOCAB = 1000000
_D = 64
_T = 128
_B = 4096
_S = 200

_NC = 2
_NS = 16
_NW = _NC * _NS
_BPW = _B // _NW
_V2 = 524288


def _tc_pair_table(emb_table):
    tt = emb_table.T
    q_blk = 4096
    n_blk = _V2 // q_blk

    def body(a_ref, b_ref, o_ref):
        o_ref[...] = jnp.concatenate([a_ref[...].T, b_ref[...].T], axis=1)

    return pl.pallas_call(
        body,
        out_shape=jax.ShapeDtypeStruct((_V2, 2 * _D), jnp.float32),
        grid=(n_blk,),
        in_specs=[
            pl.BlockSpec((_D, q_blk), lambda i: (0, i)),
            pl.BlockSpec((_D, q_blk),
                         lambda i: (0, jnp.minimum(i + n_blk,
                                                   _VOCAB // q_blk))),
        ],
        out_specs=pl.BlockSpec((q_blk, 2 * _D), lambda i: (i, 0)),
        compiler_params=pltpu.CompilerParams(
            dimension_semantics=("parallel",)),
    )(tt, tt)


def _sc_pool_body(xt_hbm, t2_hbm, z_hbm, out_hbm, idx_v, h0, h1, s0, s1,
                  r0, r1, pool_v, acc_sh, sem0, sem1):
    sid = lax.axis_index("s")
    wid = sid * _NC + lax.axis_index("c")
    base = wid * _BPW
    abase = sid * (2 * _BPW)

    pltpu.sync_copy(z_hbm, acc_sh.at[pl.ds(abase, 2 * _BPW)])

    pltpu.sync_copy(xt_hbm.at[:, pl.ds(base, _BPW)], idx_v)

    def prep(j, hv, sv):
        for k in range(_BPW // 16):
            ids = idx_v[j, pl.ds(16 * k, 16)]
            big = ids >= _V2
            hv[pl.ds(16 * k, 16)] = ids - jnp.where(big, _V2, 0)
            sv[pl.ds(16 * k, 16)] = (lax.iota(jnp.int32, 16)
                                     + (16 * k + abase)
                                     + jnp.where(big, _BPW, 0))

    prep(0, h0, s0)
    pltpu.async_copy(t2_hbm.at[h0], r0, sem0).wait()
    prep(1, h1, s1)
    pltpu.async_copy(t2_hbm.at[h1], r1, sem1)
    pltpu.sync_copy(r0, acc_sh.at[s0], add=True)

    @pl.loop(2, _S, step=2)
    def _(j):
        prep(j, h0, s0)
        pltpu.async_copy(t2_hbm.at[h0], r0, sem0)
        pltpu.make_async_copy(t2_hbm.at[h1], r1, sem1).wait()
        pltpu.sync_copy(r1, acc_sh.at[s1], add=True)
        prep(j + 1, h1, s1)
        pltpu.async_copy(t2_hbm.at[h1], r1, sem1)
        pltpu.make_async_copy(t2_hbm.at[h0], r0, sem0).wait()
        pltpu.sync_copy(r0, acc_sh.at[s0], add=True)

    pltpu.make_async_copy(t2_hbm.at[h1], r1, sem1).wait()
    pltpu.sync_copy(r1, acc_sh.at[s1], add=True)

    pltpu.sync_copy(acc_sh.at[pl.ds(abase, _BPW)], r0)
    pltpu.sync_copy(acc_sh.at[pl.ds(abase + _BPW, _BPW)], r1)

    @pl.loop(0, _BPW)
    def _(i):
        for k in range(_D // 16):
            pool_v[i, pl.ds(16 * k, 16)] = (
                r0[i, pl.ds(16 * k, 16)] + r1[i, pl.ds(_D + 16 * k, 16)])

    pltpu.sync_copy(pool_v, out_hbm.at[pl.ds(base, _BPW)])


def _sc_pool(xt, t2, zeros):
    mesh = plsc.VectorSubcoreMesh(core_axis_name="c", subcore_axis_name="s")
    return pl.kernel(
        _sc_pool_body,
        out_type=jax.ShapeDtypeStruct((_B, _D), jnp.float32),
        mesh=mesh,
        scratch_types=[
            pltpu.VMEM((_S, _BPW), jnp.int32),
            pltpu.VMEM((_BPW,), jnp.int32),
            pltpu.VMEM((_BPW,), jnp.int32),
            pltpu.VMEM((_BPW,), jnp.int32),
            pltpu.VMEM((_BPW,), jnp.int32),
            pltpu.VMEM((_BPW, 2 * _D), jnp.float32),
            pltpu.VMEM((_BPW, 2 * _D), jnp.float32),
            pltpu.VMEM((_BPW, _D), jnp.float32),
            pltpu.VMEM_SHARED((_NS * 2 * _BPW, 2 * _D), jnp.float32),
            pltpu.SemaphoreType.DMA,
            pltpu.SemaphoreType.DMA,
        ],
        compiler_params=pltpu.CompilerParams(
            use_tc_tiling_on_sc=False, needs_layout_passes=False),
    )(xt, t2, zeros)


def _mm_body(p_ref, w_ref, b_ref, o_ref):
    o_ref[...] = (
        jnp.dot(p_ref[...], w_ref[...],
                preferred_element_type=jnp.float32,
                precision=lax.Precision.HIGHEST)
        + b_ref[...]
    )


def _tc_project(pooled, W, b):
    blk = 512
    return pl.pallas_call(
        _mm_body,
        out_shape=jax.ShapeDtypeStruct((_B, _T), jnp.float32),
        grid=(_B // blk,),
        in_specs=[
            pl.BlockSpec((blk, _D), lambda i: (i, 0)),
            pl.BlockSpec((_D, _T), lambda i: (0, 0)),
            pl.BlockSpec((1, _T), lambda i: (0, 0)),
        ],
        out_specs=pl.BlockSpec((blk, _T), lambda i: (i, 0)),
    )(pooled, W, b.reshape(1, _T))


def kernel(x, emb_table, W, b):
    xt = x.T
    t2 = _tc_pair_table(emb_table)
    zeros = jnp.zeros((2 * _BPW, 2 * _D), jnp.float32)
    pooled = _sc_pool(xt, t2, zeros)
    return _tc_project(pooled, W, b)
